# Initial kernel scaffold; baseline (speedup 1.0000x reference)
import jax
import jax.numpy as jnp
from jax.experimental import pallas as pl


def kernel(enc_seq, segment_ids):
    raise NotImplementedError("write your pallas kernel here")


# speedup vs baseline: 5.6628x; 5.6628x over previous
import functools

import jax
import jax.numpy as jnp
from jax import lax
from jax.experimental import pallas as pl
from jax.experimental.pallas import tpu as pltpu
from jax.experimental.pallas import tpu_sc as plsc

_NUM_SEGMENTS = 10000
_SEG_PAD = 10240
_N_TOKENS = 320000
_D = 128
_CHUNK = 128
_NCHUNKS = _N_TOKENS // _CHUNK
_NC = 2
_NS = 16
_NW = _NC * _NS


_mesh = plsc.VectorSubcoreMesh(core_axis_name="c", subcore_axis_name="s")


@functools.partial(
    pl.kernel,
    mesh=_mesh,
    out_type=[
        jax.ShapeDtypeStruct((_NC, _SEG_PAD, _D), jnp.float32),
        jax.ShapeDtypeStruct((_NC, _SEG_PAD), jnp.float32),
    ],
    scratch_types=[
        pltpu.VMEM((2, _CHUNK), jnp.int32),
        pltpu.VMEM((_CHUNK, _D), jnp.float32),
        pltpu.VMEM((_CHUNK,), jnp.float32),
        pltpu.VMEM((64, _D), jnp.float32),
        pltpu.VMEM_SHARED((_SEG_PAD, _D), jnp.float32),
        pltpu.VMEM_SHARED((_SEG_PAD,), jnp.float32),
    ],
)
def _sc_partial(enc_hbm, ids_hbm, sums_hbm, cnts_hbm,
                idx_v, rows_v, ones_v, zero_v, acc_sh, cnt_sh):
    cid = lax.axis_index("c")
    sid = lax.axis_index("s")
    wid = cid * _NS + sid

    for j in range(_CHUNK // 16):
        ones_v[pl.ds(j * 16, 16)] = jnp.ones((16,), jnp.float32)

    def zrow(r, carry):
        for j in range(_D // 16):
            zero_v[r, pl.ds(j * 16, 16)] = jnp.zeros((16,), jnp.float32)
        return carry

    lax.fori_loop(0, 64, zrow, 0)

    rows_per_tile = _SEG_PAD // _NS
    base_row = sid * rows_per_tile

    def zacc(t, carry):
        pltpu.sync_copy(zero_v, acc_sh.at[pl.ds(base_row + t * 64, 64)])
        return carry

    lax.fori_loop(0, rows_per_tile // 64, zacc, 0)

    def zcnt(t, carry):
        pltpu.sync_copy(zero_v.at[0], cnt_sh.at[pl.ds(base_row + t * _D, _D)])
        return carry

    lax.fori_loop(0, rows_per_tile // _D, zcnt, 0)

    plsc.subcore_barrier()

    per = _NCHUNKS // _NW
    rem = _NCHUNKS - per * _NW
    base = wid * per + jnp.minimum(wid, rem)
    n_my = per + jnp.where(wid < rem, 1, 0)

    def body(i, carry):
        c = base + i
        pltpu.sync_copy(ids_hbm.at[c], idx_v.at[0])
        pltpu.sync_copy(enc_hbm.at[pl.ds(c * _CHUNK, _CHUNK)], rows_v)
        pltpu.sync_copy(rows_v, acc_sh.at[idx_v.at[0]], add=True)
        pltpu.sync_copy(ones_v, cnt_sh.at[idx_v.at[0]], add=True)
        return carry

    lax.fori_loop(0, n_my, body, 0)

    plsc.subcore_barrier()

    pltpu.sync_copy(acc_sh.at[pl.ds(base_row, rows_per_tile)],
                    sums_hbm.at[cid, pl.ds(base_row, rows_per_tile)])
    pltpu.sync_copy(cnt_sh.at[pl.ds(base_row, rows_per_tile)],
                    cnts_hbm.at[cid, pl.ds(base_row, rows_per_tile)])


def _combine(p_ref, c_ref, o_ref):
    s = p_ref[0] + p_ref[1]
    c = c_ref[0] + c_ref[1]
    c = jnp.maximum(c, 1.0)
    o_ref[...] = (s / c)[: _NUM_SEGMENTS]


@jax.jit
def _impl(enc_seq, segment_ids):
    ids2d = segment_ids.reshape(_NCHUNKS, _CHUNK)
    sums, cnts = _sc_partial(enc_seq, ids2d)
    mentions = pl.pallas_call(
        _combine,
        out_shape=jax.ShapeDtypeStruct((_NUM_SEGMENTS, _D), jnp.float32),
    )(sums, cnts.reshape(_NC, _SEG_PAD, 1))
    return mentions


def kernel(enc_seq, segment_ids):
    return _impl(enc_seq, segment_ids)


# speedup vs baseline: 9.2299x; 1.6299x over previous
import functools

import jax
import jax.numpy as jnp
from jax import lax
from jax.experimental import pallas as pl
from jax.experimental.pallas import tpu as pltpu
from jax.experimental.pallas import tpu_sc as plsc

_NUM_SEGMENTS = 10000
_SEG_PAD = 10240
_N_TOKENS = 320000
_D = 128
_SUB = 128
_BLOCK = 128
_NSUB = _BLOCK // _SUB
_NBLOCKS = _N_TOKENS // _BLOCK
_NC = 2
_NS = 16
_NW = _NC * _NS


_mesh = plsc.VectorSubcoreMesh(core_axis_name="c", subcore_axis_name="s")


@functools.partial(
    pl.kernel,
    mesh=_mesh,
    out_type=[
        jax.ShapeDtypeStruct((_NC, _SEG_PAD, _D), jnp.float32),
        jax.ShapeDtypeStruct((_NC, _SEG_PAD), jnp.float32),
    ],
    scratch_types=[
        pltpu.VMEM((2, _NSUB, _SUB), jnp.int32),
        pltpu.VMEM((2, _BLOCK, _D), jnp.float32),
        pltpu.VMEM((_SUB,), jnp.float32),
        pltpu.VMEM((32, _D), jnp.float32),
        pltpu.VMEM_SHARED((_SEG_PAD, _D), jnp.float32),
        pltpu.VMEM_SHARED((_SEG_PAD,), jnp.float32),
        pltpu.SemaphoreType.DMA((2,)),
        pltpu.SemaphoreType.DMA((2,)),
    ],
)
def _sc_partial(enc_hbm, ids_hbm, sums_hbm, cnts_hbm,
                idx_v, rows_v, ones_v, zero_v, acc_sh, cnt_sh,
                sem_rows, sem_ids):
    cid = lax.axis_index("c")
    sid = lax.axis_index("s")
    wid = cid * _NS + sid

    for j in range(_SUB // 16):
        ones_v[pl.ds(j * 16, 16)] = jnp.ones((16,), jnp.float32)

    def zrow(r, carry):
        for j in range(_D // 16):
            zero_v[r, pl.ds(j * 16, 16)] = jnp.zeros((16,), jnp.float32)
        return carry

    lax.fori_loop(0, 32, zrow, 0)

    rows_per_tile = _SEG_PAD // _NS
    base_row = sid * rows_per_tile

    def zacc(t, carry):
        pltpu.sync_copy(zero_v, acc_sh.at[pl.ds(base_row + t * 32, 32)])
        return carry

    lax.fori_loop(0, rows_per_tile // 32, zacc, 0)

    def zcnt(t, carry):
        pltpu.sync_copy(zero_v.at[0], cnt_sh.at[pl.ds(base_row + t * _D, _D)])
        return carry

    lax.fori_loop(0, rows_per_tile // _D, zcnt, 0)

    plsc.subcore_barrier()

    per = _NBLOCKS // _NW
    rem = _NBLOCKS - per * _NW
    base = wid * per + jnp.minimum(wid, rem)
    n_my = per + jnp.where(wid < rem, 1, 0)

    def _start_load(c, b):
        pltpu.async_copy(enc_hbm.at[pl.ds(c * _BLOCK, _BLOCK)], rows_v.at[b],
                         sem_rows.at[b])
        pltpu.async_copy(ids_hbm.at[c], idx_v.at[b], sem_ids.at[b])

    def _wait_load(c, b):
        pltpu.make_async_copy(enc_hbm.at[pl.ds(c * _BLOCK, _BLOCK)],
                              rows_v.at[b], sem_rows.at[b]).wait()
        pltpu.make_async_copy(ids_hbm.at[c], idx_v.at[b],
                              sem_ids.at[b]).wait()

    @pl.when(n_my > 0)
    def _prime():
        _start_load(base, 0)

    def body(i, carry):
        b = i % 2

        @pl.when(i + 1 < n_my)
        def _next():
            _start_load(base + i + 1, (i + 1) % 2)

        _wait_load(base + i, b)
        for j in range(_NSUB):
            idx_row = idx_v.at[b, j]
            pltpu.sync_copy(rows_v.at[b, pl.ds(j * _SUB, _SUB)],
                            acc_sh.at[idx_row], add=True)
            pltpu.sync_copy(ones_v, cnt_sh.at[idx_row], add=True)
        return carry

    lax.fori_loop(0, n_my, body, 0)

    plsc.subcore_barrier()

    pltpu.sync_copy(acc_sh.at[pl.ds(base_row, rows_per_tile)],
                    sums_hbm.at[cid, pl.ds(base_row, rows_per_tile)])
    pltpu.sync_copy(cnt_sh.at[pl.ds(base_row, rows_per_tile)],
                    cnts_hbm.at[cid, pl.ds(base_row, rows_per_tile)])


def _combine(p_ref, c_ref, o_ref):
    s = p_ref[0] + p_ref[1]
    c = c_ref[0] + c_ref[1]
    c = jnp.maximum(c, 1.0)
    o_ref[...] = (s / c)[: _NUM_SEGMENTS]


@jax.jit
def _impl(enc_seq, segment_ids):
    ids3d = segment_ids.reshape(_NBLOCKS, _NSUB, _SUB)
    sums, cnts = _sc_partial(enc_seq, ids3d)
    mentions = pl.pallas_call(
        _combine,
        out_shape=jax.ShapeDtypeStruct((_NUM_SEGMENTS, _D), jnp.float32),
    )(sums, cnts.reshape(_NC, _SEG_PAD, 1))
    return mentions


def kernel(enc_seq, segment_ids):
    return _impl(enc_seq, segment_ids)
